# baseline (speedup 1.0000x reference)
import functools

import jax
import jax.numpy as jnp
from jax.experimental import pallas as pl
from jax.experimental.pallas import tpu as pltpu

_N = 16
_D = 16
_K = 8
_Q = 128
_BB = 512


def _seg_consts(dtype=jnp.float32):
    l = jax.lax.broadcasted_iota(jnp.int32, (1, _N * _D), 1)
    i = l // _D
    j = l % _D
    return i, j


def _fused_body(x_ref, ew1, eb1, ew2, eb2, gw1, gb1, gw2, gb2, cbw,
                dw1, db1, dw2, db2, recon_ref, hot_ref, kk_ref):
    f32 = jnp.float32
    dot = lambda a, b: jax.lax.dot_general(
        a.astype(jnp.bfloat16), b.astype(jnp.bfloat16),
        (((1,), (1,)), ((), ())), preferred_element_type=f32)

    x = x_ref[...]
    h1 = dot(x, ew1[...]) + eb1[...]
    h1 = h1 * jax.nn.sigmoid(h1)
    enc = dot(h1, ew2[...]) + eb2[...]
    g1 = dot(enc, gw1[...]) + gb1[...]
    g1 = g1 * jax.nn.sigmoid(g1)
    gate = dot(g1, gw2[...]) + gb2[...]

    seg_i, seg_j = _seg_consts()
    pi = jax.lax.broadcasted_iota(jnp.int32, (_N, _N * _D), 0)
    T_tile = (pi == seg_j).astype(f32)
    R_rep = (pi == seg_i).astype(f32)
    Mseg = (seg_i.T == jax.lax.broadcasted_iota(
        jnp.int32, (_N * _D, _N), 1)).astype(f32)
    MsegJ = (seg_j.T == jax.lax.broadcasted_iota(
        jnp.int32, (_N * _D, _N), 1)).astype(f32)

    mm = lambda a, b: jax.lax.dot_general(
        a, b, (((1,), (0,)), ((), ())), preferred_element_type=f32,
        precision=jax.lax.Precision.HIGHEST)

    gate_t = mm(gate, T_tile)
    gate_r = mm(gate, R_rep)
    tie = (seg_j < seg_i)
    beats = jnp.where((gate_t > gate_r) | ((gate_t == gate_r) & tie), 1.0, 0.0)
    rank16 = mm(beats, Mseg)

    m = jnp.max(enc, axis=1, keepdims=True)
    e = jnp.exp(enc - m)
    seg_sum = mm(e, Mseg)
    sum_lane = mm(seg_sum, R_rep)
    khot = jnp.where(e * _D > sum_lane, 1.0, 0.0)

    seg_cnt = mm(khot, Mseg)
    selm = jnp.where(rank16 < _K, 1.0, 0.0)
    kk = jnp.clip(jnp.sum(seg_cnt * selm, axis=1, keepdims=True), 1.0, 128.0)

    rank_lane = mm(rank16, R_rep)
    hots = []
    for r in range(_K):
        xr = khot * jnp.where(rank_lane == float(r), 1.0, 0.0)
        hots.append(mm(xr, MsegJ))
    hot = jnp.concatenate(hots, axis=1)

    hot_n = hot / kk
    q = dot(hot_n, cbw[...])
    d1 = dot(q, dw1[...]) + db1[...]
    d1 = d1 * jax.nn.sigmoid(d1)
    recon = dot(d1, dw2[...]) + db2[...]

    recon_ref[...] = recon
    hot_ref[...] = hot
    kk_ref[...] = kk


@functools.partial(jax.jit, static_argnames=("interpret",))
def _run(x, ew1, eb1, ew2, eb2, gw1, gb1, gw2, gb2, cbw, dw1, db1, dw2, db2,
         interpret=False):
    b = x.shape[0]
    grid = (b // _BB,)
    full = lambda shape: pl.BlockSpec(shape, lambda i: (0,) * len(shape))
    out = pl.pallas_call(
        _fused_body,
        grid=grid,
        in_specs=[
            pl.BlockSpec((_BB, 128), lambda i: (i, 0)),
            full((256, 128)), full((1, 256)),
            full((256, 256)), full((1, 256)),
            full((256, 256)), full((1, 256)),
            full((16, 256)), full((1, 16)),
            full((64, 128)),
            full((256, 64)), full((1, 256)),
            full((128, 256)), full((1, 128)),
        ],
        out_specs=[
            pl.BlockSpec((_BB, 128), lambda i: (i, 0)),
            pl.BlockSpec((_BB, 128), lambda i: (i, 0)),
            pl.BlockSpec((_BB, 1), lambda i: (i, 0)),
        ],
        out_shape=[
            jax.ShapeDtypeStruct((b, 128), jnp.float32),
            jax.ShapeDtypeStruct((b, 128), jnp.float32),
            jax.ShapeDtypeStruct((b, 1), jnp.float32),
        ],
        interpret=interpret,
    )(x, ew1, eb1.reshape(1, -1), ew2, eb2.reshape(1, -1), gw1,
      gb1.reshape(1, -1), gw2, gb2.reshape(1, -1), cbw, dw1,
      db1.reshape(1, -1), dw2, db2.reshape(1, -1))
    return out


def kernel(x, enc_w1, enc_b1, enc_w2, enc_b2, gate_w1, gate_b1, gate_w2,
           gate_b2, cb_w, dec_w1, dec_b1, dec_w2, dec_b2):
    recon, hot, kk = _run(x, enc_w1, enc_b1, enc_w2, enc_b2, gate_w1, gate_b1,
                          gate_w2, gate_b2, cb_w, dec_w1, dec_b1, dec_w2,
                          dec_b2)
    return (recon, hot, jnp.float32(0.0), kk)


# speedup vs baseline: 1.5931x; 1.5931x over previous
import functools

import jax
import jax.numpy as jnp
from jax.experimental import pallas as pl
from jax.experimental.pallas import tpu as pltpu

_N = 16
_D = 16
_K = 8
_Q = 128
_BB = 512


def _seg_consts(dtype=jnp.float32):
    l = jax.lax.broadcasted_iota(jnp.int32, (1, _N * _D), 1)
    i = l // _D
    j = l % _D
    return i, j


def _fused_body(x_ref, ew1, eb1, ew2, eb2, gw1, gb1, gw2, gb2, cbw,
                dw1, db1, dw2, db2, recon_ref, hot_ref, kk_ref):
    f32 = jnp.float32
    dot = lambda a, b: jax.lax.dot_general(
        a.astype(jnp.bfloat16), b.astype(jnp.bfloat16),
        (((1,), (1,)), ((), ())), preferred_element_type=f32)

    x = x_ref[...]
    h1 = dot(x, ew1[...]) + eb1[...]
    h1 = h1 * jax.nn.sigmoid(h1)
    enc = dot(h1, ew2[...]) + eb2[...]
    g1 = dot(enc, gw1[...]) + gb1[...]
    g1 = g1 * jax.nn.sigmoid(g1)
    gate = dot(g1, gw2[...]) + gb2[...]

    seg_i, seg_j = _seg_consts()
    pi = jax.lax.broadcasted_iota(jnp.int32, (_N, _N * _D), 0)
    T_tile = (pi == seg_j).astype(f32)
    R_rep = (pi == seg_i).astype(f32)
    Mseg = (seg_i.T == jax.lax.broadcasted_iota(
        jnp.int32, (_N * _D, _N), 1)).astype(f32)
    MsegJ = (seg_j.T == jax.lax.broadcasted_iota(
        jnp.int32, (_N * _D, _N), 1)).astype(f32)

    mml = lambda a, b: jax.lax.dot_general(
        a, b, (((1,), (0,)), ((), ())), preferred_element_type=f32)
    mmh = lambda a, b: jax.lax.dot_general(
        a, b, (((1,), (0,)), ((), ())), preferred_element_type=f32,
        precision=jax.lax.Precision.HIGHEST)

    gate_t = pltpu.repeat(gate, _N, 1)
    gate_r = mmh(gate, R_rep)
    tie = (seg_j < seg_i)
    beats = jnp.where((gate_t > gate_r) | ((gate_t == gate_r) & tie), 1.0, 0.0)
    rank16 = mml(beats, Mseg)

    m = jnp.max(enc, axis=1, keepdims=True)
    e = jnp.exp(enc - m)
    seg_sum = mmh(e, Mseg)
    sum_lane = mmh(seg_sum, R_rep)
    khot = jnp.where(e * _D > sum_lane, 1.0, 0.0)

    seg_cnt = mml(khot, Mseg)
    selm = jnp.where(rank16 < _K, 1.0, 0.0)
    kk = jnp.clip(jnp.sum(seg_cnt * selm, axis=1, keepdims=True), 1.0, 128.0)

    rank_lane = mml(rank16, R_rep)
    hots = []
    for r in range(_K):
        xr = khot * jnp.where(rank_lane == float(r), 1.0, 0.0)
        hots.append(mml(xr, MsegJ))
    hot = jnp.concatenate(hots, axis=1)

    hot_n = hot / kk
    q = dot(hot_n, cbw[...])
    d1 = dot(q, dw1[...]) + db1[...]
    d1 = d1 * jax.nn.sigmoid(d1)
    recon = dot(d1, dw2[...]) + db2[...]

    recon_ref[...] = recon
    hot_ref[...] = hot
    kk_ref[...] = kk


@functools.partial(jax.jit, static_argnames=("interpret",))
def _run(x, ew1, eb1, ew2, eb2, gw1, gb1, gw2, gb2, cbw, dw1, db1, dw2, db2,
         interpret=False):
    b = x.shape[0]
    grid = (b // _BB,)
    full = lambda shape: pl.BlockSpec(shape, lambda i: (0,) * len(shape))
    out = pl.pallas_call(
        _fused_body,
        grid=grid,
        in_specs=[
            pl.BlockSpec((_BB, 128), lambda i: (i, 0)),
            full((256, 128)), full((1, 256)),
            full((256, 256)), full((1, 256)),
            full((256, 256)), full((1, 256)),
            full((16, 256)), full((1, 16)),
            full((64, 128)),
            full((256, 64)), full((1, 256)),
            full((128, 256)), full((1, 128)),
        ],
        out_specs=[
            pl.BlockSpec((_BB, 128), lambda i: (i, 0)),
            pl.BlockSpec((_BB, 128), lambda i: (i, 0)),
            pl.BlockSpec((_BB, 1), lambda i: (i, 0)),
        ],
        out_shape=[
            jax.ShapeDtypeStruct((b, 128), jnp.float32),
            jax.ShapeDtypeStruct((b, 128), jnp.float32),
            jax.ShapeDtypeStruct((b, 1), jnp.float32),
        ],
        interpret=interpret,
    )(x, ew1, eb1.reshape(1, -1), ew2, eb2.reshape(1, -1), gw1,
      gb1.reshape(1, -1), gw2, gb2.reshape(1, -1), cbw, dw1,
      db1.reshape(1, -1), dw2, db2.reshape(1, -1))
    return out


def kernel(x, enc_w1, enc_b1, enc_w2, enc_b2, gate_w1, gate_b1, gate_w2,
           gate_b2, cb_w, dec_w1, dec_b1, dec_w2, dec_b2):
    recon, hot, kk = _run(x, enc_w1, enc_b1, enc_w2, enc_b2, gate_w1, gate_b1,
                          gate_w2, gate_b2, cb_w, dec_w1, dec_b1, dec_w2,
                          dec_b2)
    return (recon, hot, jnp.float32(0.0), kk)
